# baseline (speedup 1.0000x reference)
import functools

import jax
import jax.numpy as jnp
from jax import lax
from jax.experimental import pallas as pl
from jax.experimental.pallas import tpu as pltpu
from jax.experimental.pallas import tpu_sc as plsc

N_SPANS = 8192
D = 128
DW = D
K = 32
P = N_SPANS * K
HID = 150
HIDP = 256
BINS_VALS = (1, 2, 3, 4, 8, 16, 32, 64)

S = 1
P_S = P // S

NC, NS = 2, 16
NW = NC * NS
ROWS_TOTAL = 2 * P_S
ROWS_PER_W = ROWS_TOTAL // NW
CHUNK = 256
NCHUNKS = ROWS_PER_W // CHUNK
KD = 2
NG = NCHUNKS // KD

T = 4096
NT = P_S // T


def _sc_gather_body(table_hbm, ids_hbm, out_hbm, idx_v, rows_v, gsem, wsem):
    wid = lax.axis_index("s") * NC + lax.axis_index("c")
    base = wid * ROWS_PER_W
    pltpu.sync_copy(ids_hbm.at[wid], idx_v)

    def group(g, _):
        descs = []
        for b in range(KD):
            d = pltpu.async_copy(
                table_hbm.at[idx_v.at[g * KD + b]], rows_v.at[b], gsem)
            descs.append(d)
        wdescs = []
        for b in range(KD):
            descs[b].wait()
            wd = pltpu.async_copy(
                rows_v.at[b],
                out_hbm.at[pl.ds(base + (g * KD + b) * CHUNK, CHUNK)],
                wsem)
            wdescs.append(wd)
        for b in range(KD):
            wdescs[b].wait()
        return 0

    lax.fori_loop(0, NG, group, 0)


@functools.cache
def _build_sc_gather():
    return functools.partial(
        pl.kernel,
        out_type=jax.ShapeDtypeStruct((ROWS_TOTAL, DW), jnp.int32),
        mesh=plsc.VectorSubcoreMesh(
            core_axis_name="c", subcore_axis_name="s",
            num_cores=NC, num_subcores=NS),
        scratch_types=[
            pltpu.VMEM((NCHUNKS, CHUNK), jnp.int32),
            pltpu.VMEM((KD, CHUNK, DW), jnp.int32),
            pltpu.SemaphoreType.DMA,
            pltpu.SemaphoreType.DMA,
        ],
        compiler_params=pltpu.CompilerParams(use_tc_tiling_on_sc=False),
    )(_sc_gather_body)


def _sc_gather(table_packed, ids3):
    return _build_sc_gather()(table_packed, ids3)


def _tc_mlp_body(gi_ref, gj_ref, dist_ref, spk_ref, bins_ref,
                 w1a_ref, w1b_ref, w1c_ref, otw_ref, w2_ref, b2_ref,
                 w3_ref, b3_ref, out_ref):
    bf16 = jnp.bfloat16
    gi = lax.bitcast_convert_type(gi_ref[...], jnp.float32).astype(bf16)
    gj = lax.bitcast_convert_type(gj_ref[...], jnp.float32).astype(bf16)
    gij = gi * gj
    col = lax.broadcasted_iota(jnp.int32, (T, 16), 1)
    distb = jnp.broadcast_to(dist_ref[...], (T, 16))
    spkb = jnp.broadcast_to(spk_ref[...], (T, 16))
    binsb = jnp.broadcast_to(bins_ref[...], (T, 16))
    m = jnp.where(
        col < 8, (distb > binsb).astype(jnp.float32),
        jnp.where(col < 11, (spkb == (col - 8)).astype(jnp.float32),
                  jnp.where(col == 11, 1.0, 0.0)))
    acc = jnp.dot(gi, w1a_ref[...], preferred_element_type=jnp.float32)
    acc += jnp.dot(gj, w1b_ref[...], preferred_element_type=jnp.float32)
    acc += jnp.dot(gij, w1c_ref[...], preferred_element_type=jnp.float32)
    acc += jnp.dot(m, otw_ref[...], preferred_element_type=jnp.float32)
    h = jnp.maximum(acc, 0.0).astype(bf16)
    h2 = jnp.dot(h, w2_ref[...], preferred_element_type=jnp.float32)
    h2 = jnp.maximum(h2 + b2_ref[...], 0.0).astype(bf16)
    r = jnp.dot(h2, w3_ref[...], preferred_element_type=jnp.float32)
    r = r + b3_ref[...]
    out_ref[...] = r[:, :2]


def _tc_mlp(gathered, dist2, spk2, binspad, w1s, otw, w2p, b2p, w3p, b3p):
    full = lambda shape: pl.BlockSpec(shape, lambda i: (0, 0))
    return pl.pallas_call(
        _tc_mlp_body,
        grid=(NT,),
        in_specs=[
            pl.BlockSpec((T, DW), lambda i: (i, 0)),
            pl.BlockSpec((T, DW), lambda i: (i + NT, 0)),
            pl.BlockSpec((T, 1), lambda i: (i, 0)),
            pl.BlockSpec((T, 1), lambda i: (i, 0)),
            full((1, 16)),
        ] + [full((D, HIDP))] * 3 + [
            full((16, HIDP)),
            full((HIDP, HIDP)), full((1, HIDP)),
            full((HIDP, 128)), full((1, 128)),
        ],
        out_specs=pl.BlockSpec((T, 2), lambda i: (i, 0)),
        out_shape=jax.ShapeDtypeStruct((P_S, 2), jnp.float32),
        compiler_params=pltpu.CompilerParams(
            dimension_semantics=("arbitrary",)),
    )(gathered, gathered, dist2, spk2, binspad, *w1s, otw, w2p, b2p, w3p,
      b3p)


def kernel(span_features, mention_ids, antecedent_ids, distances, speakers,
           dist_table, spk_table, W1, b1, W2, b2, W3, b3, epsilon):
    f32 = jnp.float32
    bf16 = jnp.bfloat16
    w1a = jnp.zeros((D, HIDP), f32).at[:, :HID].set(W1[0:D]).astype(bf16)
    w1b = jnp.zeros((D, HIDP), f32).at[:, :HID].set(W1[D:2 * D]).astype(bf16)
    w1c = jnp.zeros((D, HIDP), f32).at[:, :HID].set(
        W1[2 * D:3 * D]).astype(bf16)
    w1s = (w1a, w1b, w1c)
    w1d = W1[3 * D:3 * D + 20]
    w1sp = W1[3 * D + 20:3 * D + 40]
    deltas = (dist_table[1:9] - dist_table[0:8]) @ w1d
    spkrows = spk_table @ w1sp
    const_row = dist_table[0] @ w1d + b1
    otw = jnp.zeros((16, HIDP), f32)
    otw = otw.at[0:8, :HID].set(deltas)
    otw = otw.at[8:11, :HID].set(spkrows)
    otw = otw.at[11, :HID].set(const_row)
    w2p = jnp.zeros((HIDP, HIDP), f32).at[:HID, :HID].set(W2).astype(bf16)
    b2p = jnp.zeros((1, HIDP), f32).at[0, :HID].set(b2)
    w3p = jnp.zeros((HIDP, 128), f32).at[:HID, :2].set(W3).astype(bf16)
    b3p = jnp.zeros((1, 128), f32).at[0, :2].set(b3)
    binspad = jnp.full((1, 16), 2**30, jnp.int32).at[0, :8].set(
        jnp.array(BINS_VALS, jnp.int32))

    sfp = lax.bitcast_convert_type(span_features, jnp.int32)

    mids = mention_ids.astype(jnp.int32).reshape(S, P_S)
    aids = antecedent_ids.astype(jnp.int32).reshape(S, P_S)
    dist2 = distances.astype(jnp.int32).reshape(S, P_S, 1)
    spk2 = speakers.astype(jnp.int32).reshape(S, P_S, 1)
    gathers = []
    for s in range(S):
        ids3 = jnp.concatenate([mids[s], aids[s]]).reshape(
            NW, NCHUNKS, CHUNK)
        gathers.append(_sc_gather(sfp, ids3))
    outs = [
        _tc_mlp(gathers[s], dist2[s], spk2[s], binspad, w1s, otw, w2p,
                b2p, w3p, b3p)
        for s in range(S)
    ]
    scores = jnp.concatenate(outs)

    scores = scores.reshape(N_SPANS, K, 2)
    eps = jnp.broadcast_to(epsilon.reshape(1, 1, 2), (N_SPANS, 1, 2))
    return jnp.concatenate([scores, eps], axis=1)


# speedup vs baseline: 1.0478x; 1.0478x over previous
import functools

import jax
import jax.numpy as jnp
from jax import lax
from jax.experimental import pallas as pl
from jax.experimental.pallas import tpu as pltpu
from jax.experimental.pallas import tpu_sc as plsc

N_SPANS = 8192
D = 128
DW = D
K = 32
P = N_SPANS * K
HID = 150
HIDP = 256
BINS_VALS = (1, 2, 3, 4, 8, 16, 32, 64)

S = 1
P_S = P // S

NC, NS = 2, 16
NW = NC * NS
ROWS_TOTAL = 2 * P_S
ROWS_PER_W = ROWS_TOTAL // NW
CHUNK = 128
NCHUNKS = ROWS_PER_W // CHUNK
KD = 4
NG = NCHUNKS // KD

T = 4096
NT = P_S // T


def _sc_gather_body(table_hbm, ids_hbm, out_hbm, idx_v, rows_v, gsem, wsem):
    wid = lax.axis_index("s") * NC + lax.axis_index("c")
    base = wid * ROWS_PER_W
    pltpu.sync_copy(ids_hbm.at[wid], idx_v)

    def group(g, _):
        descs = []
        for b in range(KD):
            d = pltpu.async_copy(
                table_hbm.at[idx_v.at[g * KD + b]], rows_v.at[b], gsem)
            descs.append(d)
        wdescs = []
        for b in range(KD):
            descs[b].wait()
            wd = pltpu.async_copy(
                rows_v.at[b],
                out_hbm.at[pl.ds(base + (g * KD + b) * CHUNK, CHUNK)],
                wsem)
            wdescs.append(wd)
        for b in range(KD):
            wdescs[b].wait()
        return 0

    lax.fori_loop(0, NG, group, 0)


@functools.cache
def _build_sc_gather():
    return functools.partial(
        pl.kernel,
        out_type=jax.ShapeDtypeStruct((ROWS_TOTAL, DW), jnp.int32),
        mesh=plsc.VectorSubcoreMesh(
            core_axis_name="c", subcore_axis_name="s",
            num_cores=NC, num_subcores=NS),
        scratch_types=[
            pltpu.VMEM((NCHUNKS, CHUNK), jnp.int32),
            pltpu.VMEM((KD, CHUNK, DW), jnp.int32),
            pltpu.SemaphoreType.DMA,
            pltpu.SemaphoreType.DMA,
        ],
        compiler_params=pltpu.CompilerParams(use_tc_tiling_on_sc=False),
    )(_sc_gather_body)


def _sc_gather(table_packed, ids3):
    return _build_sc_gather()(table_packed, ids3)


def _tc_mlp_body(gi_ref, gj_ref, dist_ref, spk_ref, bins_ref,
                 w1_ref, otw_ref, w2_ref, w3_ref, out_ref):
    bf16 = jnp.bfloat16
    gi = lax.bitcast_convert_type(gi_ref[...], jnp.float32).astype(bf16)
    gj = lax.bitcast_convert_type(gj_ref[...], jnp.float32).astype(bf16)
    gij = gi * gj
    col = lax.broadcasted_iota(jnp.int32, (T, 16), 1)
    distb = jnp.broadcast_to(dist_ref[...], (T, 16))
    spkb = jnp.broadcast_to(spk_ref[...], (T, 16))
    binsb = jnp.broadcast_to(bins_ref[...], (T, 16))
    m = jnp.where(
        col < 8, (distb > binsb).astype(jnp.float32),
        jnp.where(col < 11, (spkb == (col - 8)).astype(jnp.float32),
                  jnp.where(col == 11, 1.0, 0.0))).astype(bf16)
    x = jnp.concatenate([gi, gj, gij], axis=1)
    acc = jnp.dot(x, w1_ref[...], preferred_element_type=jnp.float32)
    acc += jnp.dot(m, otw_ref[...], preferred_element_type=jnp.float32)
    h = jnp.maximum(acc, 0.0).astype(bf16)
    h2 = jnp.dot(h, w2_ref[...], preferred_element_type=jnp.float32)
    h2 = jnp.maximum(h2, 0.0).astype(bf16)
    r = jnp.dot(h2, w3_ref[...], preferred_element_type=jnp.float32)
    out_ref[...] = r[:, :2]


def _tc_mlp(gathered, dist2, spk2, binspad, w1f, otwp, w2p, w3p):
    full = lambda shape: pl.BlockSpec(shape, lambda i: (0, 0))
    return pl.pallas_call(
        _tc_mlp_body,
        grid=(NT,),
        in_specs=[
            pl.BlockSpec((T, DW), lambda i: (i, 0)),
            pl.BlockSpec((T, DW), lambda i: (i + NT, 0)),
            pl.BlockSpec((T, 1), lambda i: (i, 0)),
            pl.BlockSpec((T, 1), lambda i: (i, 0)),
            full((1, 16)),
            full((3 * D, HIDP)),
            full((16, HIDP)),
            full((HIDP, HIDP)),
            full((HIDP, 128)),
        ],
        out_specs=pl.BlockSpec((T, 2), lambda i: (i, 0)),
        out_shape=jax.ShapeDtypeStruct((P_S, 2), jnp.float32),
        compiler_params=pltpu.CompilerParams(
            dimension_semantics=("arbitrary",)),
    )(gathered, gathered, dist2, spk2, binspad, w1f, otwp, w2p, w3p)


def kernel(span_features, mention_ids, antecedent_ids, distances, speakers,
           dist_table, spk_table, W1, b1, W2, b2, W3, b3, epsilon):
    f32 = jnp.float32
    bf16 = jnp.bfloat16
    w1d = W1[3 * D:3 * D + 20]
    w1sp = W1[3 * D + 20:3 * D + 40]
    deltas = (dist_table[1:9] - dist_table[0:8]) @ w1d
    spkrows = spk_table @ w1sp
    const_row = dist_table[0] @ w1d + b1
    w1f = jnp.zeros((3 * D, HIDP), f32).at[:, :HID].set(
        W1[0:3 * D]).astype(bf16)
    otwp = jnp.zeros((16, HIDP), f32)
    otwp = otwp.at[0:8, :HID].set(deltas)
    otwp = otwp.at[8:11, :HID].set(spkrows)
    otwp = otwp.at[11, :HID].set(const_row)
    otwp = otwp.at[11, HIDP - 1].set(1.0)
    otwp = otwp.astype(bf16)
    w2p = jnp.zeros((HIDP, HIDP), f32).at[:HID, :HID].set(W2)
    w2p = w2p.at[HIDP - 1, :HID].set(b2)
    w2p = w2p.at[HIDP - 1, HIDP - 1].set(1.0)
    w2p = w2p.astype(bf16)
    w3p = jnp.zeros((HIDP, 128), f32).at[:HID, :2].set(W3)
    w3p = w3p.at[HIDP - 1, :2].set(b3)
    w3p = w3p.astype(bf16)
    binspad = jnp.full((1, 16), 2**30, jnp.int32).at[0, :8].set(
        jnp.array(BINS_VALS, jnp.int32))

    sfp = lax.bitcast_convert_type(span_features, jnp.int32)

    mids = mention_ids.astype(jnp.int32).reshape(S, P_S)
    aids = antecedent_ids.astype(jnp.int32).reshape(S, P_S)
    dist2 = distances.astype(jnp.int32).reshape(S, P_S, 1)
    spk2 = speakers.astype(jnp.int32).reshape(S, P_S, 1)
    gathers = []
    for s in range(S):
        ids3 = jnp.concatenate([mids[s], aids[s]]).reshape(
            NW, NCHUNKS, CHUNK)
        gathers.append(_sc_gather(sfp, ids3))
    outs = [
        _tc_mlp(gathers[s], dist2[s], spk2[s], binspad, w1f, otwp, w2p,
                w3p)
        for s in range(S)
    ]
    scores = jnp.concatenate(outs)

    scores = scores.reshape(N_SPANS, K, 2)
    eps = jnp.broadcast_to(epsilon.reshape(1, 1, 2), (N_SPANS, 1, 2))
    return jnp.concatenate([scores, eps], axis=1)


# speedup vs baseline: 1.0752x; 1.0261x over previous
import functools

import jax
import jax.numpy as jnp
from jax import lax
from jax.experimental import pallas as pl
from jax.experimental.pallas import tpu as pltpu
from jax.experimental.pallas import tpu_sc as plsc

N_SPANS = 8192
D = 128
DW = D
K = 32
P = N_SPANS * K
HID = 150
HIDP = 256
BINS_VALS = (1, 2, 3, 4, 8, 16, 32, 64)

S = 1
P_S = P // S

NC, NS = 2, 16
NW = NC * NS
ROWS_TOTAL = 2 * P_S
ROWS_PER_W = ROWS_TOTAL // NW
CHUNK = 128
NCHUNKS = ROWS_PER_W // CHUNK
KD = 4
NG = NCHUNKS // KD

T = 4096
NT = P_S // T


def _sc_gather_body(table_hbm, ids_hbm, out_hbm, idx_v, rows_v, gsem, wsem):
    wid = lax.axis_index("s") * NC + lax.axis_index("c")
    base = wid * ROWS_PER_W
    pltpu.sync_copy(ids_hbm.at[wid], idx_v)

    def group(g, _):
        descs = []
        for b in range(KD):
            d = pltpu.async_copy(
                table_hbm.at[idx_v.at[g * KD + b]], rows_v.at[b], gsem)
            descs.append(d)
        wdescs = []
        for b in range(KD):
            descs[b].wait()
            wd = pltpu.async_copy(
                rows_v.at[b],
                out_hbm.at[pl.ds(base + (g * KD + b) * CHUNK, CHUNK)],
                wsem)
            wdescs.append(wd)
        for b in range(KD):
            wdescs[b].wait()
        return 0

    lax.fori_loop(0, NG, group, 0)


@functools.cache
def _build_sc_gather():
    return functools.partial(
        pl.kernel,
        out_type=jax.ShapeDtypeStruct((ROWS_TOTAL, DW), jnp.int32),
        mesh=plsc.VectorSubcoreMesh(
            core_axis_name="c", subcore_axis_name="s",
            num_cores=NC, num_subcores=NS),
        scratch_types=[
            pltpu.VMEM((NCHUNKS, CHUNK), jnp.int32),
            pltpu.VMEM((KD, CHUNK, DW), jnp.int32),
            pltpu.SemaphoreType.DMA,
            pltpu.SemaphoreType.DMA,
        ],
        compiler_params=pltpu.CompilerParams(use_tc_tiling_on_sc=False),
    )(_sc_gather_body)


def _sc_gather(table_packed, ids3):
    return _build_sc_gather()(table_packed, ids3)


def _tc_mlp_body(gi_ref, gj_ref, dist_ref, spk_ref, bins_ref,
                 w1_ref, w2_ref, w3_ref, out_ref):
    bf16 = jnp.bfloat16
    gi = lax.bitcast_convert_type(gi_ref[...], jnp.float32).astype(bf16)
    gj = lax.bitcast_convert_type(gj_ref[...], jnp.float32).astype(bf16)
    gij = gi * gj
    col = lax.broadcasted_iota(jnp.int32, (T, 16), 1)
    distb = jnp.broadcast_to(dist_ref[...], (T, 16))
    spkb = jnp.broadcast_to(spk_ref[...], (T, 16))
    binsb = jnp.broadcast_to(bins_ref[...], (T, 16))
    m = jnp.where(
        col < 8, (distb > binsb).astype(jnp.float32),
        jnp.where(col < 11, (spkb == (col - 8)).astype(jnp.float32),
                  jnp.where(col == 11, 1.0, 0.0))).astype(bf16)
    x = jnp.concatenate([gi, gj, gij, m], axis=1)
    acc = jnp.dot(x, w1_ref[...], preferred_element_type=jnp.float32)
    h = jnp.maximum(acc, 0.0).astype(bf16)
    h2 = jnp.dot(h, w2_ref[...], preferred_element_type=jnp.float32)
    h2 = jnp.maximum(h2, 0.0).astype(bf16)
    r = jnp.dot(h2, w3_ref[...], preferred_element_type=jnp.float32)
    out_ref[...] = r[:, :2]


def _tc_mlp(gathered, dist2, spk2, binspad, w1f, w2p, w3p):
    full = lambda shape: pl.BlockSpec(shape, lambda i: (0, 0))
    return pl.pallas_call(
        _tc_mlp_body,
        grid=(NT,),
        in_specs=[
            pl.BlockSpec((T, DW), lambda i: (i, 0)),
            pl.BlockSpec((T, DW), lambda i: (i + NT, 0)),
            pl.BlockSpec((T, 1), lambda i: (i, 0)),
            pl.BlockSpec((T, 1), lambda i: (i, 0)),
            full((1, 16)),
            full((3 * D + 16, HIDP)),
            full((HIDP, HIDP)),
            full((HIDP, 128)),
        ],
        out_specs=pl.BlockSpec((T, 2), lambda i: (i, 0)),
        out_shape=jax.ShapeDtypeStruct((P_S, 2), jnp.float32),
        compiler_params=pltpu.CompilerParams(
            dimension_semantics=("arbitrary",)),
    )(gathered, gathered, dist2, spk2, binspad, w1f, w2p, w3p)


def kernel(span_features, mention_ids, antecedent_ids, distances, speakers,
           dist_table, spk_table, W1, b1, W2, b2, W3, b3, epsilon):
    f32 = jnp.float32
    bf16 = jnp.bfloat16
    w1d = W1[3 * D:3 * D + 20]
    w1sp = W1[3 * D + 20:3 * D + 40]
    deltas = (dist_table[1:9] - dist_table[0:8]) @ w1d
    spkrows = spk_table @ w1sp
    const_row = dist_table[0] @ w1d + b1
    w1f = jnp.zeros((3 * D + 16, HIDP), f32)
    w1f = w1f.at[0:3 * D, :HID].set(W1[0:3 * D])
    w1f = w1f.at[3 * D:3 * D + 8, :HID].set(deltas)
    w1f = w1f.at[3 * D + 8:3 * D + 11, :HID].set(spkrows)
    w1f = w1f.at[3 * D + 11, :HID].set(const_row)
    w1f = w1f.at[3 * D + 11, HIDP - 1].set(1.0)
    w1f = w1f.astype(bf16)
    w2p = jnp.zeros((HIDP, HIDP), f32).at[:HID, :HID].set(W2)
    w2p = w2p.at[HIDP - 1, :HID].set(b2)
    w2p = w2p.at[HIDP - 1, HIDP - 1].set(1.0)
    w2p = w2p.astype(bf16)
    w3p = jnp.zeros((HIDP, 128), f32).at[:HID, :2].set(W3)
    w3p = w3p.at[HIDP - 1, :2].set(b3)
    w3p = w3p.astype(bf16)
    binspad = jnp.full((1, 16), 2**30, jnp.int32).at[0, :8].set(
        jnp.array(BINS_VALS, jnp.int32))

    sfp = lax.bitcast_convert_type(span_features, jnp.int32)

    mids = mention_ids.astype(jnp.int32).reshape(S, P_S)
    aids = antecedent_ids.astype(jnp.int32).reshape(S, P_S)
    dist2 = distances.astype(jnp.int32).reshape(S, P_S, 1)
    spk2 = speakers.astype(jnp.int32).reshape(S, P_S, 1)
    gathers = []
    for s in range(S):
        ids3 = jnp.concatenate([mids[s], aids[s]]).reshape(
            NW, NCHUNKS, CHUNK)
        gathers.append(_sc_gather(sfp, ids3))
    outs = [
        _tc_mlp(gathers[s], dist2[s], spk2[s], binspad, w1f, w2p, w3p)
        for s in range(S)
    ]
    scores = jnp.concatenate(outs)

    scores = scores.reshape(N_SPANS, K, 2)
    eps = jnp.broadcast_to(epsilon.reshape(1, 1, 2), (N_SPANS, 1, 2))
    return jnp.concatenate([scores, eps], axis=1)
